# Initial kernel scaffold; baseline (speedup 1.0000x reference)
import jax
import jax.numpy as jnp
from jax.experimental import pallas as pl


def kernel(tensor, values, indices):
    raise NotImplementedError("write your pallas kernel here")


# speedup vs baseline: 1.3858x; 1.3858x over previous
import functools

import jax
import jax.numpy as jnp
from jax import lax
from jax.experimental import pallas as pl
from jax.experimental.pallas import tpu as pltpu
from jax.experimental.pallas import tpu_sc as plsc

_SHAPE = (4096, 16384)
_N = _SHAPE[0] * _SHAPE[1]
_K = 671088

_C = 528
_NW = 32
_G = _K // _C
_GPT = 40
_LAST = _G - (_NW - 1) * _GPT
_NB = 6
_LAG_G = 2
_LAG_S = 4

_mesh = plsc.VectorSubcoreMesh(core_axis_name="c", subcore_axis_name="s")


@functools.partial(
    pl.kernel,
    mesh=_mesh,
    scratch_types=[
        [pltpu.VMEM((_C,), jnp.int32) for _ in range(_NB)],
        [pltpu.VMEM((_C,), jnp.float32) for _ in range(_NB)],
        [pltpu.VMEM((_C,), jnp.float32) for _ in range(_NB)],
        [pltpu.SemaphoreType.DMA for _ in range(_NB)],
        [pltpu.SemaphoreType.DMA for _ in range(_NB)],
        [pltpu.SemaphoreType.DMA for _ in range(_NB)],
    ],
)
def _sc_scatter_add(out_ref, idx_hbm, val_hbm, ibufs, vbufs, gbufs, lsems,
                    gsems, ssems):
    wid = lax.axis_index("s") * 2 + lax.axis_index("c")
    base = wid * _GPT * _C
    nc = jnp.where(wid < _NW - 1, _GPT, _LAST)

    def guarded(j, fn):
        if j < _LAST:
            fn(j)
        else:
            pl.when(nc == _GPT)(lambda: fn(j))

    def start_l(j):
        b = j % _NB
        off = pl.ds(base + j * _C, _C)
        pltpu.async_copy(idx_hbm.at[off], ibufs[b], lsems[b])
        pltpu.async_copy(val_hbm.at[off], vbufs[b], lsems[b])

    def wait_l(j):
        b = j % _NB
        off = pl.ds(base + j * _C, _C)
        pltpu.make_async_copy(idx_hbm.at[off], ibufs[b], lsems[b]).wait()
        pltpu.make_async_copy(val_hbm.at[off], vbufs[b], lsems[b]).wait()

    def start_g(j):
        b = j % _NB
        pltpu.async_copy(out_ref.at[ibufs[b]], gbufs[b], gsems[b])

    def wait_g(j):
        b = j % _NB
        pltpu.make_async_copy(out_ref.at[ibufs[b]], gbufs[b],
                              gsems[b]).wait()

    def add_vals(j):
        b = j % _NB
        for i in range(_C // 16):
            sl = pl.ds(i * 16, 16)
            gbufs[b][sl] = gbufs[b][sl] + vbufs[b][sl]

    def start_s(j):
        b = j % _NB
        pltpu.async_copy(gbufs[b], out_ref.at[ibufs[b]], ssems[b])

    def wait_s(j):
        b = j % _NB
        pltpu.make_async_copy(gbufs[b], out_ref.at[ibufs[b]],
                              ssems[b]).wait()

    for t in range(_GPT + _LAG_S):
        if t < _GPT:
            if t >= _NB:
                guarded(t - _NB, wait_s)
            guarded(t, start_l)
        jg = t - _LAG_G
        if 0 <= jg < _GPT:
            guarded(jg, wait_l)
            guarded(jg, start_g)
        js = t - _LAG_S
        if 0 <= js < _GPT:
            guarded(js, wait_g)
            guarded(js, add_vals)
            guarded(js, start_s)

    for j in range(_GPT - _NB, _GPT):
        guarded(j, wait_s)


def kernel(tensor, values, indices):
    flat = tensor.reshape(-1)
    idx = indices.astype(jnp.int32)
    out_ref = jax.new_ref(flat)
    _sc_scatter_add(out_ref, idx, values)
    return out_ref[...].reshape(_SHAPE)


# speedup vs baseline: 1.7316x; 1.2495x over previous
import functools

import jax
import jax.numpy as jnp
from jax import lax
from jax.experimental import pallas as pl
from jax.experimental.pallas import tpu as pltpu
from jax.experimental.pallas import tpu_sc as plsc

_SHAPE = (4096, 16384)
_N = _SHAPE[0] * _SHAPE[1]
_K = 671088

_NW = 32
_DR = _N // _NW
_B = 16384
_NBLK = _DR // _B
_NBUF = 4
_W = _B + 4096

_mesh = plsc.VectorSubcoreMesh(core_axis_name="c", subcore_axis_name="s")


@functools.partial(
    pl.kernel,
    mesh=_mesh,
    compiler_params=pltpu.CompilerParams(needs_layout_passes=False),
    out_type=jax.ShapeDtypeStruct((_N,), jnp.float32),
    scratch_types=[
        [pltpu.VMEM((_B,), jnp.float32) for _ in range(_NBUF)],
        pltpu.VMEM((_W + 16,), jnp.int32),
        pltpu.VMEM((_W + 16,), jnp.float32),
        pltpu.VMEM((48,), jnp.int32),
        [pltpu.SemaphoreType.DMA for _ in range(_NBUF)],
        [pltpu.SemaphoreType.DMA for _ in range(_NBUF)],
    ],
)
def _sc_apply(tensor_hbm, idx_hbm, val_hbm, bounds_hbm, out_hbm,
              dbufs, idx_win, val_win, bounds_v, lsems, ssems):
    wid = lax.axis_index("s") * 2 + lax.axis_index("c")
    tile_base = wid * _DR

    pltpu.sync_copy(bounds_hbm, bounds_v)
    lane_w = jnp.full((16,), wid, dtype=jnp.int32)
    s_lo = jnp.max(plsc.load_gather(bounds_v, [lane_w]))
    s_hi = jnp.max(plsc.load_gather(bounds_v, [lane_w + 1]))

    iota = lax.iota(jnp.int32, 16)

    def start_load(k):
        b = k % _NBUF
        pltpu.async_copy(tensor_hbm.at[pl.ds(tile_base + k * _B, _B)],
                         dbufs[b], lsems[b])

    def process_block(k, b, p, wbase):
        blo = tile_base + k * _B
        bhi = blo + _B

        need = (p - wbase) > (_W - _B - 16)
        wb_new = pl.multiple_of(
            jnp.where(need, jnp.minimum(p - lax.rem(p, 8), _K - _W), wbase),
            8)

        @pl.when(need)
        def _():
            pltpu.sync_copy(idx_hbm.at[pl.ds(wb_new, _W)],
                            idx_win.at[pl.ds(0, _W)])
            pltpu.sync_copy(val_hbm.at[pl.ds(wb_new, _W)],
                            val_win.at[pl.ds(0, _W)])

        def body(carry):
            p_c, _ = carry
            off = p_c - wb_new
            head = lax.rem(off, 8)
            off_al = pl.multiple_of(off - head, 8)
            iv = idx_win[pl.ds(off_al, 16)]
            vv = val_win[pl.ds(off_al, 16)]
            q = (wb_new + off_al) + iota
            valid = (q >= p_c) & (q < s_hi) & (iv < bhi)
            plsc.addupdate_scatter(dbufs[b], [iv - blo], vv, mask=valid)
            n = jnp.sum(valid.astype(jnp.int32))
            return p_c + n, n == (16 - head)

        p_new, _ = lax.while_loop(lambda c: c[1], body, (p, True))
        return p_new, wb_new

    for k0 in range(2):
        start_load(k0)

    def one_block(k, b, carry):
        p, wbase = carry
        pltpu.make_async_copy(
            tensor_hbm.at[pl.ds(tile_base + k * _B, _B)], dbufs[b],
            lsems[b]).wait()
        p, wbase = process_block(k, b, p, wbase)
        pltpu.async_copy(dbufs[b], out_hbm.at[pl.ds(tile_base + k * _B, _B)],
                         ssems[b])
        kn = k + 2
        bn = (b + 2) % _NBUF

        @pl.when(kn < _NBLK)
        def _():
            @pl.when(kn >= _NBUF)
            def _():
                ko = kn - _NBUF
                pltpu.make_async_copy(
                    dbufs[bn],
                    out_hbm.at[pl.ds(tile_base + ko * _B, _B)],
                    ssems[bn]).wait()
            pltpu.async_copy(
                tensor_hbm.at[pl.ds(tile_base + kn * _B, _B)],
                dbufs[bn], lsems[bn])

        return p, wbase

    @pl.loop(0, _NBLK // _NBUF,
             init_carry=(s_lo, jnp.int32(-2 * _W)))
    def _outer(k4, carry):
        for b in range(_NBUF):
            k = k4 * _NBUF + b
            carry = one_block(k, b, carry)
        return carry

    for kk in range(_NBLK - _NBUF, _NBLK):
        b = kk % _NBUF
        pltpu.make_async_copy(
            dbufs[b], out_hbm.at[pl.ds(tile_base + kk * _B, _B)],
            ssems[b]).wait()


def kernel(tensor, values, indices):
    flat = tensor.reshape(-1)
    idx = indices.astype(jnp.int32)
    edges = jnp.arange(0, _N + 1, _DR, dtype=jnp.int32)
    bounds = jnp.searchsorted(idx, edges).astype(jnp.int32)
    bounds = jnp.pad(bounds, (0, 48 - bounds.shape[0]))
    out = _sc_apply(flat, idx, values, bounds)
    return out.reshape(_SHAPE)
